# Initial kernel scaffold; baseline (speedup 1.0000x reference)
import jax
import jax.numpy as jnp
from jax.experimental import pallas as pl


def kernel(x, W, b):
    raise NotImplementedError("write your pallas kernel here")


# speedup vs baseline: 3.6551x; 3.6551x over previous
import functools

import jax
import jax.numpy as jnp
from jax.experimental import pallas as pl

_S = 196
_P = 256


def _sort_pool_body(x_ref, w_ref, b_ref, o_ref):
    v = x_ref[0]
    n, l = v.shape
    iota = jax.lax.broadcasted_iota(jnp.int32, (n, l), 0)

    k = 2
    while k <= n:
        d = k // 2
        while d >= 1:
            up = jnp.concatenate([v[d:], v[:d]], axis=0)
            dn = jnp.concatenate([v[n - d:], v[:n - d]], axis=0)
            is_hi = (iota & d) != 0
            partner = jnp.where(is_hi, dn, up)
            dir_flag = (iota & k) != 0
            want_max = is_hi == dir_flag
            v = jnp.where(want_max, jnp.maximum(v, partner),
                          jnp.minimum(v, partner))
            d //= 2
        k *= 2

    v = jnp.where(iota < _S, v, 0.0)
    o_ref[0, 0, :] = jnp.sum(v * w_ref[...], axis=0) + b_ref[...]


@jax.jit
def kernel(x, W, b):
    B, C, H, Wd = x.shape
    S = H * Wd
    L = 128

    xt = jnp.transpose(x.reshape(B, C, S), (0, 2, 1))
    xt = jnp.pad(xt, ((0, 0), (0, _P - S), (0, 0)),
                 constant_values=-jnp.inf)
    Wt = jnp.pad(W[:, 0, :].T, ((0, _P - S), (0, 0)))

    grid = (B, C // L)
    out = pl.pallas_call(
        _sort_pool_body,
        grid=grid,
        in_specs=[
            pl.BlockSpec((1, _P, L), lambda i, j: (i, 0, j)),
            pl.BlockSpec((_P, L), lambda i, j: (0, j)),
            pl.BlockSpec((L,), lambda i, j: (j,)),
        ],
        out_specs=pl.BlockSpec((1, 1, L), lambda i, j: (i, 0, j)),
        out_shape=jax.ShapeDtypeStruct((B, 1, C), jnp.float32),
    )(xt, Wt, b)
    return out.reshape(B, C)


# speedup vs baseline: 5.5967x; 1.5312x over previous
import jax
import jax.numpy as jnp
from jax.experimental import pallas as pl

_S = 196
_P = 256
_LOG = 8
_NEG = -1e30


def _bitrev(i, bits=_LOG):
    r = 0
    for _ in range(bits):
        r = (r << 1) | (i & 1)
        i >>= 1
    return r


def _cmpex_aligned(v, D, B):
    n = v.shape[0]
    pieces = []
    for t in range(0, n // D, 2):
        E = v[t * D:(t + 1) * D]
        O = v[(t + 1) * D:(t + 2) * D]
        mn = jnp.minimum(E, O)
        mx = jnp.maximum(E, O)
        if B is None:
            newE, newO = mx, mn
        elif B >= 8:
            ep, op = [], []
            for u in range(0, D, B):
                if (u & B) == 0:
                    ep.append(mx[u:u + B])
                    op.append(mn[u:u + B])
                else:
                    ep.append(mn[u:u + B])
                    op.append(mx[u:u + B])
            newE = jnp.concatenate(ep, axis=0)
            newO = jnp.concatenate(op, axis=0)
        else:
            dm = (jax.lax.broadcasted_iota(jnp.int32, E.shape, 0) & B) != 0
            newE = jnp.where(dm, mn, mx)
            newO = jnp.where(dm, mx, mn)
        pieces += [newE, newO]
    return jnp.concatenate(pieces, axis=0)


def _cmpex_small(v, D, B):
    n = v.shape[0]
    iota = jax.lax.broadcasted_iota(jnp.int32, v.shape, 0)
    up = jnp.concatenate([v[D:], v[:D]], axis=0)
    dn = jnp.concatenate([v[n - D:], v[:n - D]], axis=0)
    is_hi = (iota & D) != 0
    partner = jnp.where(is_hi, dn, up)
    dirm = (iota & B) != 0
    want_max = is_hi == dirm
    return jnp.where(want_max, jnp.maximum(v, partner),
                     jnp.minimum(v, partner))


def _sort_pool_body(x_ref, w_ref, b_ref, o_ref):
    v = x_ref[0]

    for a in range(1, _LOG + 1):
        B = None if a == _LOG else (1 << (_LOG - 1 - a))
        for m in range(a - 1, -1, -1):
            D = 1 << (_LOG - 1 - m)
            if D >= 8:
                v = _cmpex_aligned(v, D, B)
            else:
                v = _cmpex_small(v, D, B if B is not None else 0)

    o_ref[0, 0, :] = jnp.sum(v * w_ref[...], axis=0) + b_ref[...]


@jax.jit
def kernel(x, W, b):
    B, C, H, Wd = x.shape
    S = H * Wd
    L = 128

    xt = jnp.transpose(x.reshape(B, C, S), (0, 2, 1))
    xt = jnp.pad(xt, ((0, 0), (0, _P - S), (0, 0)), constant_values=_NEG)

    rev = jnp.array([_bitrev(i) for i in range(_P)], dtype=jnp.int32)
    Wt = jnp.pad(W[:, 0, :].T, ((0, _P - S), (0, 0)))[rev, :]

    grid = (B, C // L)
    out = pl.pallas_call(
        _sort_pool_body,
        grid=grid,
        in_specs=[
            pl.BlockSpec((1, _P, L), lambda i, j: (i, 0, j)),
            pl.BlockSpec((_P, L), lambda i, j: (0, j)),
            pl.BlockSpec((L,), lambda i, j: (j,)),
        ],
        out_specs=pl.BlockSpec((1, 1, L), lambda i, j: (i, 0, j)),
        out_shape=jax.ShapeDtypeStruct((B, 1, C), jnp.float32),
    )(xt, Wt, b)
    return out.reshape(B, C)


# speedup vs baseline: 5.8775x; 1.0502x over previous
import jax
import jax.numpy as jnp
from jax.experimental import pallas as pl

_S = 196
_P = 256
_LOG = 8
_NEG = -1e30


def _bitrev(i, bits=_LOG):
    r = 0
    for _ in range(bits):
        r = (r << 1) | (i & 1)
        i >>= 1
    return r


def _cmpex_aligned(v, D, B):
    n = v.shape[0]
    pieces = []
    for t in range(0, n // D, 2):
        E = v[t * D:(t + 1) * D]
        O = v[(t + 1) * D:(t + 2) * D]
        mn = jnp.minimum(E, O)
        mx = jnp.maximum(E, O)
        if B is None:
            newE, newO = mx, mn
        elif B >= 8:
            ep, op = [], []
            for u in range(0, D, B):
                if (u & B) == 0:
                    ep.append(mx[u:u + B])
                    op.append(mn[u:u + B])
                else:
                    ep.append(mn[u:u + B])
                    op.append(mx[u:u + B])
            newE = jnp.concatenate(ep, axis=0)
            newO = jnp.concatenate(op, axis=0)
        else:
            dm = (jax.lax.broadcasted_iota(jnp.int32, E.shape, 0) & B) != 0
            newE = jnp.where(dm, mn, mx)
            newO = jnp.where(dm, mx, mn)
        pieces += [newE, newO]
    return jnp.concatenate(pieces, axis=0)


def _cmpex_small(v, D, B):
    n = v.shape[0]
    iota = jax.lax.broadcasted_iota(jnp.int32, v.shape, 0)
    up = jnp.concatenate([v[D:], v[:D]], axis=0)
    dn = jnp.concatenate([v[n - D:], v[:n - D]], axis=0)
    is_hi = (iota & D) != 0
    partner = jnp.where(is_hi, dn, up)
    dirm = (iota & B) != 0
    want_max = is_hi == dirm
    return jnp.where(want_max, jnp.maximum(v, partner),
                     jnp.minimum(v, partner))


def _sort_pool_body(x_ref, w_ref, b_ref, o_ref):
    xb = x_ref[0]
    pad = jnp.full((xb.shape[0], _P - _S), _NEG, dtype=xb.dtype)
    v = jnp.concatenate([xb, pad], axis=1).T

    for a in range(1, _LOG + 1):
        B = None if a == _LOG else (1 << (_LOG - 1 - a))
        for m in range(a - 1, -1, -1):
            D = 1 << (_LOG - 1 - m)
            if D >= 8:
                v = _cmpex_aligned(v, D, B)
            else:
                v = _cmpex_small(v, D, B if B is not None else 0)

    o_ref[0, 0, :] = jnp.sum(v * w_ref[...], axis=0) + b_ref[...]


@jax.jit
def kernel(x, W, b):
    B, C, H, Wd = x.shape
    S = H * Wd
    L = 128

    xt = x.reshape(B, C, S)

    rev = jnp.array([_bitrev(i) for i in range(_P)], dtype=jnp.int32)
    Wt = jnp.pad(W[:, 0, :].T, ((0, _P - S), (0, 0)))[rev, :]

    grid = (B, C // L)
    out = pl.pallas_call(
        _sort_pool_body,
        grid=grid,
        in_specs=[
            pl.BlockSpec((1, L, S), lambda i, j: (i, j, 0)),
            pl.BlockSpec((_P, L), lambda i, j: (0, j)),
            pl.BlockSpec((L,), lambda i, j: (j,)),
        ],
        out_specs=pl.BlockSpec((1, 1, L), lambda i, j: (i, 0, j)),
        out_shape=jax.ShapeDtypeStruct((B, 1, C), jnp.float32),
    )(xt, Wt, b)
    return out.reshape(B, C)


# speedup vs baseline: 7.4651x; 1.2701x over previous
import jax
import jax.numpy as jnp
from jax.experimental import pallas as pl

_S = 196
_P = 256
_LOG = 8
_NEG = -1e30


def _bitrev(i, bits=_LOG):
    r = 0
    for _ in range(bits):
        r = (r << 1) | (i & 1)
        i >>= 1
    return r


def _cmpex_aligned(v, D, B):
    n = v.shape[0]
    pieces = []
    for t in range(0, n // D, 2):
        E = v[t * D:(t + 1) * D]
        O = v[(t + 1) * D:(t + 2) * D]
        mn = jnp.minimum(E, O)
        mx = jnp.maximum(E, O)
        if B is None:
            newE, newO = mx, mn
        elif B >= 8:
            ep, op = [], []
            for u in range(0, D, B):
                if (u & B) == 0:
                    ep.append(mx[u:u + B])
                    op.append(mn[u:u + B])
                else:
                    ep.append(mn[u:u + B])
                    op.append(mx[u:u + B])
            newE = jnp.concatenate(ep, axis=0)
            newO = jnp.concatenate(op, axis=0)
        else:
            dm = (jax.lax.broadcasted_iota(jnp.int32, E.shape, 0) & B) != 0
            newE = jnp.where(dm, mn, mx)
            newO = jnp.where(dm, mx, mn)
        pieces += [newE, newO]
    return jnp.concatenate(pieces, axis=0)


def _cmpex_small(v, D, B):
    n = v.shape[0]
    iota = jax.lax.broadcasted_iota(jnp.int32, v.shape, 0)
    up = jnp.concatenate([v[D:], v[:D]], axis=0)
    dn = jnp.concatenate([v[n - D:], v[:n - D]], axis=0)
    is_hi = (iota & D) != 0
    partner = jnp.where(is_hi, dn, up)
    dirm = (iota & B) != 0
    want_max = is_hi == dirm
    return jnp.where(want_max, jnp.maximum(v, partner),
                     jnp.minimum(v, partner))


def _sort_pool_body(x_ref, w_ref, b_ref, o_ref):
    xb = x_ref[0]
    pad = jnp.full((xb.shape[0], _P - _S), _NEG, dtype=xb.dtype)
    v = jnp.concatenate([xb, pad], axis=1).T

    for a in range(1, _LOG + 1):
        B = None if a == _LOG else (1 << (_LOG - 1 - a))
        for m in range(a - 1, -1, -1):
            D = 1 << (_LOG - 1 - m)
            if D >= 8:
                v = _cmpex_aligned(v, D, B)
            else:
                v = _cmpex_small(v, D, B if B is not None else 0)

    o_ref[0, 0, :] = jnp.sum(v * w_ref[...], axis=0) + b_ref[...]


@jax.jit
def kernel(x, W, b):
    B, C, H, Wd = x.shape
    S = H * Wd
    L = 256

    xt = x.reshape(B, C, S)

    rev = jnp.array([_bitrev(i) for i in range(_P)], dtype=jnp.int32)
    Wt = jnp.pad(W[:, 0, :].T, ((0, _P - S), (0, 0)))[rev, :]

    grid = (B, C // L)
    out = pl.pallas_call(
        _sort_pool_body,
        grid=grid,
        in_specs=[
            pl.BlockSpec((1, L, S), lambda i, j: (i, j, 0)),
            pl.BlockSpec((_P, L), lambda i, j: (0, j)),
            pl.BlockSpec((L,), lambda i, j: (j,)),
        ],
        out_specs=pl.BlockSpec((1, 1, L), lambda i, j: (i, 0, j)),
        out_shape=jax.ShapeDtypeStruct((B, 1, C), jnp.float32),
    )(xt, Wt, b)
    return out.reshape(B, C)


# speedup vs baseline: 7.8729x; 1.0546x over previous
import jax
import jax.numpy as jnp
from jax.experimental import pallas as pl

_S = 196
_P = 256
_LOG = 8
_NEG = -1e30


def _bitrev(i, bits=_LOG):
    r = 0
    for _ in range(bits):
        r = (r << 1) | (i & 1)
        i >>= 1
    return r


def _cmpex_aligned(v, D, B):
    n = v.shape[0]
    pieces = []
    for t in range(0, n // D, 2):
        E = v[t * D:(t + 1) * D]
        O = v[(t + 1) * D:(t + 2) * D]
        mn = jnp.minimum(E, O)
        mx = jnp.maximum(E, O)
        if B is None:
            newE, newO = mx, mn
        elif B >= 8:
            ep, op = [], []
            for u in range(0, D, B):
                if (u & B) == 0:
                    ep.append(mx[u:u + B])
                    op.append(mn[u:u + B])
                else:
                    ep.append(mn[u:u + B])
                    op.append(mx[u:u + B])
            newE = jnp.concatenate(ep, axis=0)
            newO = jnp.concatenate(op, axis=0)
        else:
            dm = (jax.lax.broadcasted_iota(jnp.int32, E.shape, 0) & B) != 0
            newE = jnp.where(dm, mn, mx)
            newO = jnp.where(dm, mx, mn)
        pieces += [newE, newO]
    return jnp.concatenate(pieces, axis=0)


def _cmpex_small(v, D, B):
    n = v.shape[0]
    iota = jax.lax.broadcasted_iota(jnp.int32, v.shape, 0)
    up = jnp.concatenate([v[D:], v[:D]], axis=0)
    dn = jnp.concatenate([v[n - D:], v[:n - D]], axis=0)
    is_hi = (iota & D) != 0
    partner = jnp.where(is_hi, dn, up)
    dirm = (iota & B) != 0
    want_max = is_hi == dirm
    return jnp.where(want_max, jnp.maximum(v, partner),
                     jnp.minimum(v, partner))


def _sort_pool_body(x_ref, w_ref, b_ref, o_ref):
    xb = x_ref[0]
    pad = jnp.full((xb.shape[0], _P - _S), _NEG, dtype=xb.dtype)
    v = jnp.concatenate([xb, pad], axis=1).T

    for a in range(1, _LOG + 1):
        B = None if a == _LOG else (1 << (_LOG - 1 - a))
        for m in range(a - 1, -1, -1):
            D = 1 << (_LOG - 1 - m)
            if D >= 8:
                v = _cmpex_aligned(v, D, B)
            else:
                v = _cmpex_small(v, D, B if B is not None else 0)

    o_ref[0, 0, :] = jnp.sum(v * w_ref[...], axis=0) + b_ref[...]


@jax.jit
def kernel(x, W, b):
    B, C, H, Wd = x.shape
    S = H * Wd
    L = 768

    xt = x.reshape(B, C, S)

    rev = jnp.array([_bitrev(i) for i in range(_P)], dtype=jnp.int32)
    Wt = jnp.pad(W[:, 0, :].T, ((0, _P - S), (0, 0)))[rev, :]

    grid = (B, C // L)
    out = pl.pallas_call(
        _sort_pool_body,
        grid=grid,
        in_specs=[
            pl.BlockSpec((1, L, S), lambda i, j: (i, j, 0)),
            pl.BlockSpec((_P, L), lambda i, j: (0, j)),
            pl.BlockSpec((L,), lambda i, j: (j,)),
        ],
        out_specs=pl.BlockSpec((1, 1, L), lambda i, j: (i, 0, j)),
        out_shape=jax.ShapeDtypeStruct((B, 1, C), jnp.float32),
    )(xt, Wt, b)
    return out.reshape(B, C)
